# baseline (speedup 1.0000x reference)
import functools
import jax
import jax.numpy as jnp
import numpy as np
from jax import lax
from jax.experimental import pallas as pl
from jax.experimental.pallas import tpu as pltpu

_CHANS, _SEQ, _D = 16, 4096, 768

_PERM_CACHE = None


def _perm() -> np.ndarray:
    global _PERM_CACHE
    if _PERM_CACHE is None:
        with jax.ensure_compile_time_eval():
            _PERM_CACHE = np.asarray(
                jax.random.permutation(jax.random.key(42), _SEQ)).astype(np.int32)
    return _PERM_CACHE


def _body(idx_ref, x_ref, o_ref):
    o_ref[...] = x_ref[...]


@functools.cache
def _build():
    grid_spec = pltpu.PrefetchScalarGridSpec(
        num_scalar_prefetch=1,
        grid=(_SEQ,),
        in_specs=[pl.BlockSpec((_CHANS, 1, 1, _D),
                               lambda s, idx: (0, idx[s], 0, 0))],
        out_specs=pl.BlockSpec((_CHANS, 1, 1, _D), lambda s, idx: (0, s, 0, 0)),
    )
    return pl.pallas_call(
        _body,
        grid_spec=grid_spec,
        out_shape=jax.ShapeDtypeStruct((_CHANS, _SEQ, 1, _D), jnp.float32),
    )


def kernel(x):
    out = _build()(jnp.asarray(_perm()), x.reshape(_CHANS, _SEQ, 1, _D))
    return out.reshape(_CHANS, _SEQ, _D)
